# Initial kernel scaffold; baseline (speedup 1.0000x reference)
import jax
import jax.numpy as jnp
from jax.experimental import pallas as pl


def kernel(x, edge_index, batch, W1, b1, W2, b2, Wf1, bf1, Wf2, bf2):
    raise NotImplementedError("write your pallas kernel here")


# speedup vs baseline: 4.3871x; 4.3871x over previous
import jax
import jax.numpy as jnp
from jax import lax
from jax.experimental import pallas as pl
from jax.experimental.pallas import tpu as pltpu
from jax.experimental.pallas import tpu_sc as plsc

NC = 2
NS = 16
NW = NC * NS
L = 16
SHIFT = 14
IMASK = (1 << SHIFT) - 1
KB = 128
CHUNK = 2048
NEG = float("-inf")
GRAPHS = 8

_SC_PARAMS = pltpu.CompilerParams(needs_layout_passes=False)


def _popcount(m):
    return plsc.all_reduce_population_count(m)[0]


def _bin_edges(packed, rows_w):
    ep = packed.shape[0]
    n_chunks = ep // CHUNK
    cap = ep
    mesh = plsc.VectorSubcoreMesh(
        core_axis_name="c", subcore_axis_name="s", num_cores=NC, num_subcores=NS
    )

    def body(packed_hbm, binned_hbm, counts_hbm, inbuf, pbuf, cntb, sem_in):
        w = lax.axis_index("s") * NC + lax.axis_index("c")
        base = w * rows_w
        rbase = w * cap

        pltpu.async_copy(
            packed_hbm.at[pl.ds(0, CHUNK)], inbuf.at[pl.ds(0, CHUNK)], sem_in
        )

        def chunk_body(c, carry):
            p, nf = carry
            b = (c % 2) * CHUNK
            pltpu.make_async_copy(
                packed_hbm.at[pl.ds(0, CHUNK)], inbuf.at[pl.ds(b, CHUNK)], sem_in
            ).wait()

            @pl.when(c + 1 < n_chunks)
            def _():
                b2 = ((c + 1) % 2) * CHUNK
                pltpu.async_copy(
                    packed_hbm.at[pl.ds((c + 1) * CHUNK, CHUNK)],
                    inbuf.at[pl.ds(b2, CHUNK)],
                    sem_in,
                )

            def scan_body(i, carry):
                p, nf = carry
                v = inbuf[pl.ds(b + i * L, L)]
                dl = (v >> SHIFT) - base
                m = (dl >= 0) & (dl < rows_w)
                plsc.store_compressed(pbuf.at[pl.ds(p, L)], v, mask=m)
                p = p + _popcount(m)
                full = p >= KB

                @pl.when(full)
                def _():
                    pltpu.sync_copy(
                        pbuf.at[pl.ds(0, KB)],
                        binned_hbm.at[pl.ds(rbase + nf * KB, KB)],
                    )
                    tail = pbuf[pl.ds(KB, L)]
                    pbuf[pl.ds(0, L)] = tail

                p = jnp.where(full, p - KB, p)
                nf = jnp.where(full, nf + 1, nf)
                return p, nf

            return lax.fori_loop(0, CHUNK // L, scan_body, (p, nf))

        p, nf = lax.fori_loop(0, n_chunks, chunk_body, (jnp.int32(0), jnp.int32(0)))

        pltpu.sync_copy(
            pbuf.at[pl.ds(0, KB)],
            binned_hbm.at[pl.ds(rbase + nf * KB, KB)],
        )
        count = nf * KB + p
        cntb[pl.ds(0, L)] = jnp.full((L,), 0, jnp.int32) + count
        pltpu.sync_copy(cntb.at[pl.ds(0, L)], counts_hbm.at[pl.ds(w * 128, L)])

    return pl.kernel(
        body,
        out_type=(
            jax.ShapeDtypeStruct((NW * cap,), jnp.int32),
            jax.ShapeDtypeStruct((NW * 128,), jnp.int32),
        ),
        mesh=mesh,
        compiler_params=_SC_PARAMS,
        scratch_types=[
            pltpu.VMEM((2 * CHUNK,), jnp.int32),
            pltpu.VMEM((KB + L,), jnp.int32),
            pltpu.VMEM((L,), jnp.int32),
            pltpu.SemaphoreType.DMA,
        ],
    )(packed)


def _seg_max(binned, counts, h, n_pad, rows_w, cap):
    n, d = h.shape
    nr = d // L
    mesh = plsc.VectorSubcoreMesh(
        core_axis_name="c", subcore_axis_name="s", num_cores=NC, num_subcores=NS
    )

    def body(binned_hbm, counts_hbm, h_hbm, m_hbm, pbuf, idxb, cntb, rows, acc, sem_in, sem_g):
        w = lax.axis_index("s") * NC + lax.axis_index("c")
        base = w * rows_w
        rbase = w * cap
        neg = jnp.full((L,), NEG, jnp.float32)

        pltpu.sync_copy(counts_hbm.at[pl.ds(w * 128, L)], cntb)
        count = cntb[pl.ds(0, L)][0]
        n_blocks = (count + KB - 1) >> 7

        def init_body(i, carry):
            for r in range(nr):
                acc[i, pl.ds(r * L, L)] = neg
            return carry

        lax.fori_loop(0, rows_w, init_body, 0)

        def prep(g, slot):
            off = slot * KB
            pltpu.make_async_copy(
                binned_hbm.at[pl.ds(rbase, KB)], pbuf.at[pl.ds(off, KB)], sem_in
            ).wait()

            for j in range(KB // L):
                v = pbuf[pl.ds(off + j * L, L)]
                idxb[pl.ds(off + j * L, L)] = jnp.minimum(v & IMASK, n - 1)
            pltpu.async_copy(
                h_hbm.at[idxb.at[pl.ds(off, KB)]],
                rows.at[pl.ds(slot * KB, KB)],
                sem_g,
            )

        def fetch(g, slot):
            pltpu.async_copy(
                binned_hbm.at[pl.ds(rbase + g * KB, KB)],
                pbuf.at[pl.ds(slot * KB, KB)],
                sem_in,
            )

        def fold(g, slot):
            pltpu.make_async_copy(
                h_hbm.at[idxb.at[pl.ds(slot * KB, KB)]],
                rows.at[pl.ds(slot * KB, KB)],
                sem_g,
            ).wait()
            nvalid = jnp.minimum(count - g * KB, KB)

            def fold_e(e, carry):
                v = pbuf[pl.ds(slot * KB + e, L)][0]
                dl = (v >> SHIFT) - base
                for r in range(nr):
                    sl = pl.ds(r * L, L)
                    acc[dl, sl] = jnp.maximum(
                        acc[dl, sl], rows[slot * KB + e, pl.ds(r * L, L)]
                    )
                return carry

            lax.fori_loop(0, nvalid, fold_e, 0)

        @pl.when(n_blocks > 0)
        def _():
            fetch(jnp.int32(0), jnp.int32(0))
            prep(jnp.int32(0), jnp.int32(0))

            def block_body(g, carry):
                slot = g % 2
                nslot = (g + 1) % 2

                @pl.when(g + 1 < n_blocks)
                def _():
                    fetch(g + 1, nslot)
                    prep(g + 1, nslot)

                fold(g, slot)
                return carry

            lax.fori_loop(0, n_blocks, block_body, 0)

        pltpu.sync_copy(acc, m_hbm.at[pl.ds(base, rows_w)])

    return pl.kernel(
        body,
        out_type=jax.ShapeDtypeStruct((n_pad, d), jnp.float32),
        mesh=mesh,
        compiler_params=_SC_PARAMS,
        scratch_types=[
            pltpu.VMEM((2 * KB + L,), jnp.int32),
            pltpu.VMEM((2 * KB,), jnp.int32),
            pltpu.VMEM((L,), jnp.int32),
            pltpu.VMEM((2 * KB, d), jnp.float32),
            pltpu.VMEM((rows_w, d), jnp.float32),
            pltpu.SemaphoreType.DMA,
            pltpu.SemaphoreType.DMA,
        ],
    )(binned, counts, h)


def _layer(h, m_pad, wa_t, wb_t, b2d, rb):
    n, d = h.shape
    grid = (n // rb,)

    def body(h_ref, m_ref, wa_ref, wb_ref, b_ref, o_ref):
        hv = h_ref[...]
        mv = m_ref[...]
        aggr = jnp.where(mv == NEG, 0.0, mv - hv)
        acc = jnp.dot(hv, wa_ref[...], preferred_element_type=jnp.float32)
        acc = acc + jnp.dot(aggr, wb_ref[...], preferred_element_type=jnp.float32)
        o_ref[...] = jnp.maximum(acc + b_ref[...], 0.0)

    return pl.pallas_call(
        body,
        grid=grid,
        in_specs=[
            pl.BlockSpec((rb, d), lambda i: (i, 0)),
            pl.BlockSpec((rb, d), lambda i: (i, 0)),
            pl.BlockSpec((d, d), lambda i: (0, 0)),
            pl.BlockSpec((d, d), lambda i: (0, 0)),
            pl.BlockSpec((1, d), lambda i: (0, 0)),
        ],
        out_specs=pl.BlockSpec((rb, d), lambda i: (i, 0)),
        out_shape=jax.ShapeDtypeStruct((n, d), jnp.float32),
    )(h, m_pad, wa_t, wb_t, b2d)


def _pool_head(h2, batch2, w1_t, b1_2d, w2_t, b2_2d, rb):
    n, d = h2.shape
    nb = n // rb
    dh = w1_t.shape[1]
    nc = w2_t.shape[1]

    def body(h_ref, b_ref, w1_ref, b1_ref, w2_ref, b2_ref, o_ref, acc_ref):
        pid = pl.program_id(0)

        @pl.when(pid == 0)
        def _():
            acc_ref[...] = jnp.full((GRAPHS, d), NEG, jnp.float32)

        bv = b_ref[...]
        hv = h_ref[...]
        for g in range(GRAPHS):
            sel = jnp.where(bv == g, hv, NEG)
            red = jnp.max(sel, axis=0, keepdims=True)
            acc_ref[pl.ds(g, 1), :] = jnp.maximum(acc_ref[pl.ds(g, 1), :], red)

        @pl.when(pid == nb - 1)
        def _():
            pooled = acc_ref[...]
            pooled = jnp.where(pooled == NEG, 0.0, pooled)
            z1 = jnp.dot(pooled, w1_ref[...], preferred_element_type=jnp.float32)
            z1 = jnp.maximum(z1 + b1_ref[...], 0.0)
            z = jnp.dot(z1, w2_ref[...], preferred_element_type=jnp.float32)
            z = z + b2_ref[...]
            zm = jnp.max(z, axis=1, keepdims=True)
            lse = jnp.log(jnp.sum(jnp.exp(z - zm), axis=1, keepdims=True)) + zm
            o_ref[...] = z - lse

    return pl.pallas_call(
        body,
        grid=(nb,),
        in_specs=[
            pl.BlockSpec((rb, d), lambda i: (i, 0)),
            pl.BlockSpec((rb, 1), lambda i: (i, 0)),
            pl.BlockSpec((d, dh), lambda i: (0, 0)),
            pl.BlockSpec((1, dh), lambda i: (0, 0)),
            pl.BlockSpec((dh, nc), lambda i: (0, 0)),
            pl.BlockSpec((1, nc), lambda i: (0, 0)),
        ],
        out_specs=pl.BlockSpec((GRAPHS, nc), lambda i: (0, 0)),
        out_shape=jax.ShapeDtypeStruct((GRAPHS, nc), jnp.float32),
        scratch_shapes=[pltpu.VMEM((GRAPHS, d), jnp.float32)],
    )(h2, batch2, w1_t, b1_2d, w2_t, b2_2d)


def kernel(x, edge_index, batch, W1, b1, W2, b2, Wf1, bf1, Wf2, bf2):
    n, d = x.shape
    e = edge_index.shape[1]
    assert n <= IMASK + 1
    rows_w = (-(-n // NW) + 7) // 8 * 8
    n_pad = rows_w * NW
    rb = 2000

    src, dst = edge_index[0], edge_index[1]
    s2 = jnp.concatenate([src, dst])
    d2 = jnp.concatenate([dst, src])
    packed = jnp.bitwise_or(s2, d2 << SHIFT).astype(jnp.int32)
    ep = -(-2 * e // CHUNK) * CHUNK
    if ep > 2 * e:
        sentinel = jnp.full((ep - 2 * e,), IMASK << SHIFT, jnp.int32)
        packed = jnp.concatenate([packed, sentinel])

    w1a_t = W1[:, :d].T
    w1b_t = W1[:, d:].T
    w2a_t = W2[:, :d].T
    w2b_t = W2[:, d:].T

    binned, counts = _bin_edges(packed, rows_w)
    m1 = _seg_max(binned, counts, x, n_pad, rows_w, ep)
    h1 = _layer(x, m1, w1a_t, w1b_t, b1.reshape(1, d), rb)
    m2 = _seg_max(binned, counts, h1, n_pad, rows_w, ep)
    h2 = _layer(h1, m2, w2a_t, w2b_t, b2.reshape(1, d), rb)

    batch2 = batch.reshape(n, 1)
    return _pool_head(
        h2, batch2, Wf1.T, bf1.reshape(1, -1), Wf2.T, bf2.reshape(1, -1), rb
    )


# speedup vs baseline: 5.2569x; 1.1983x over previous
import jax
import jax.numpy as jnp
from jax import lax
from jax.experimental import pallas as pl
from jax.experimental.pallas import tpu as pltpu
from jax.experimental.pallas import tpu_sc as plsc

NC = 2
NS = 16
NW = NC * NS
L = 16
SHIFT = 14
IMASK = (1 << SHIFT) - 1
KB = 128
CHUNK = 2048
NEG = float("-inf")
GRAPHS = 8

_SC_PARAMS = pltpu.CompilerParams(needs_layout_passes=False)


def _popcount(m):
    return plsc.all_reduce_population_count(m)[0]


def _bin_edges(packed, rows_w):
    ep = packed.shape[0]
    n_chunks = ep // CHUNK
    cap = ep
    mesh = plsc.VectorSubcoreMesh(
        core_axis_name="c", subcore_axis_name="s", num_cores=NC, num_subcores=NS
    )

    def body(packed_hbm, binned_hbm, counts_hbm, inbuf, pbuf, cntb, sem_in):
        w = lax.axis_index("s") * NC + lax.axis_index("c")
        base = w * rows_w
        rbase = w * cap

        pltpu.async_copy(
            packed_hbm.at[pl.ds(0, CHUNK)], inbuf.at[pl.ds(0, CHUNK)], sem_in
        )

        def chunk_body(c, carry):
            p, nf = carry
            b = (c % 2) * CHUNK
            pltpu.make_async_copy(
                packed_hbm.at[pl.ds(0, CHUNK)], inbuf.at[pl.ds(b, CHUNK)], sem_in
            ).wait()

            @pl.when(c + 1 < n_chunks)
            def _():
                b2 = ((c + 1) % 2) * CHUNK
                pltpu.async_copy(
                    packed_hbm.at[pl.ds((c + 1) * CHUNK, CHUNK)],
                    inbuf.at[pl.ds(b2, CHUNK)],
                    sem_in,
                )

            def scan_body(i, carry):
                p, nf = carry
                v = inbuf[pl.ds(b + i * L, L)]
                dl = (v >> SHIFT) - base
                m = (dl >= 0) & (dl < rows_w)
                plsc.store_compressed(pbuf.at[pl.ds(p, L)], v, mask=m)
                p = p + _popcount(m)
                full = p >= KB

                @pl.when(full)
                def _():
                    pltpu.sync_copy(
                        pbuf.at[pl.ds(0, KB)],
                        binned_hbm.at[pl.ds(rbase + nf * KB, KB)],
                    )
                    tail = pbuf[pl.ds(KB, L)]
                    pbuf[pl.ds(0, L)] = tail

                p = jnp.where(full, p - KB, p)
                nf = jnp.where(full, nf + 1, nf)
                return p, nf

            return lax.fori_loop(0, CHUNK // L, scan_body, (p, nf))

        p, nf = lax.fori_loop(0, n_chunks, chunk_body, (jnp.int32(0), jnp.int32(0)))

        pltpu.sync_copy(
            pbuf.at[pl.ds(0, KB)],
            binned_hbm.at[pl.ds(rbase + nf * KB, KB)],
        )
        count = nf * KB + p
        cntb[pl.ds(0, L)] = jnp.full((L,), 0, jnp.int32) + count
        pltpu.sync_copy(cntb.at[pl.ds(0, L)], counts_hbm.at[pl.ds(w * 128, L)])

    return pl.kernel(
        body,
        out_type=(
            jax.ShapeDtypeStruct((NW * cap,), jnp.int32),
            jax.ShapeDtypeStruct((NW * 128,), jnp.int32),
        ),
        mesh=mesh,
        compiler_params=_SC_PARAMS,
        scratch_types=[
            pltpu.VMEM((2 * CHUNK,), jnp.int32),
            pltpu.VMEM((KB + L,), jnp.int32),
            pltpu.VMEM((L,), jnp.int32),
            pltpu.SemaphoreType.DMA,
        ],
    )(packed)


def _seg_max(binned, counts, h, n_pad, rows_w, cap):
    n, d = h.shape
    nr = d // L
    mesh = plsc.VectorSubcoreMesh(
        core_axis_name="c", subcore_axis_name="s", num_cores=NC, num_subcores=NS
    )

    ns = 4

    def body(binned_hbm, counts_hbm, h_hbm, m_hbm, pbuf, idxb, cntb, rows, acc,
             sem_in, sg0, sg1, sg2, sg3):
        w = lax.axis_index("s") * NC + lax.axis_index("c")
        base = w * rows_w
        rbase = w * cap
        neg = jnp.full((L,), NEG, jnp.float32)
        sems = (sg0, sg1, sg2, sg3)

        pltpu.sync_copy(counts_hbm.at[pl.ds(w * 128, L)], cntb)
        count = cntb[pl.ds(0, L)][0]
        n_blocks = (count + KB - 1) >> 7

        def init_body(i, carry):
            for r in range(nr):
                acc[i, pl.ds(r * L, L)] = neg
            return carry

        lax.fori_loop(0, rows_w, init_body, 0)

        def prep(g, s):
            off = s * KB
            pltpu.async_copy(
                binned_hbm.at[pl.ds(rbase + g * KB, KB)],
                pbuf.at[pl.ds(off, KB)],
                sem_in,
            )
            pltpu.make_async_copy(
                binned_hbm.at[pl.ds(rbase, KB)], pbuf.at[pl.ds(off, KB)], sem_in
            ).wait()
            for j in range(KB // L):
                v = pbuf[pl.ds(off + j * L, L)]
                idxb[pl.ds(off + j * L, L)] = jnp.minimum(v & IMASK, n - 1)
            pltpu.async_copy(
                h_hbm.at[idxb.at[pl.ds(off, KB)]],
                rows.at[pl.ds(off, KB)],
                sems[s],
            )

        def wait_gather(s):
            pltpu.make_async_copy(
                h_hbm.at[idxb.at[pl.ds(s * KB, KB)]],
                rows.at[pl.ds(s * KB, KB)],
                sems[s],
            ).wait()

        def fold_full(s):
            off = s * KB

            def fold_grp(jb, carry):
                vv = pbuf[pl.ds(off + jb * L, L)]
                dlv = (vv >> SHIFT) - base
                for je in range(L):
                    dl = dlv[je]
                    e = off + jb * L + je
                    for r in range(nr):
                        sl = pl.ds(r * L, L)
                        acc[dl, sl] = jnp.maximum(
                            acc[dl, sl], rows[e, pl.ds(r * L, L)]
                        )
                return carry

            lax.fori_loop(0, KB // L, fold_grp, 0)

        def fold_partial(g, s):
            nvalid = count - g * KB

            def fold_e(e, carry):
                v = pbuf[pl.ds(s * KB + e, L)][0]
                dl = (v >> SHIFT) - base
                for r in range(nr):
                    sl = pl.ds(r * L, L)
                    acc[dl, sl] = jnp.maximum(
                        acc[dl, sl], rows[s * KB + e, pl.ds(r * L, L)]
                    )
                return carry

            lax.fori_loop(0, nvalid, fold_e, 0)

        for k in range(ns):
            @pl.when(k < n_blocks)
            def _(k=k):
                prep(jnp.int32(k), k)

        def block_body(g, carry):
            slot = g % ns
            for s in range(ns):
                @pl.when(slot == s)
                def _(s=s):
                    wait_gather(s)

                    @pl.when(g < n_blocks - 1)
                    def _():
                        fold_full(s)

                    @pl.when(g == n_blocks - 1)
                    def _():
                        fold_partial(g, s)

                    @pl.when(g + ns < n_blocks)
                    def _():
                        prep(g + ns, s)

            return carry

        lax.fori_loop(0, n_blocks, block_body, 0)

        pltpu.sync_copy(acc, m_hbm.at[pl.ds(base, rows_w)])

    return pl.kernel(
        body,
        out_type=jax.ShapeDtypeStruct((n_pad, d), jnp.float32),
        mesh=mesh,
        compiler_params=_SC_PARAMS,
        scratch_types=[
            pltpu.VMEM((ns * KB + L,), jnp.int32),
            pltpu.VMEM((ns * KB,), jnp.int32),
            pltpu.VMEM((L,), jnp.int32),
            pltpu.VMEM((ns * KB, d), jnp.float32),
            pltpu.VMEM((rows_w, d), jnp.float32),
            pltpu.SemaphoreType.DMA,
            pltpu.SemaphoreType.DMA,
            pltpu.SemaphoreType.DMA,
            pltpu.SemaphoreType.DMA,
            pltpu.SemaphoreType.DMA,
        ],
    )(binned, counts, h)


def _layer(h, m_pad, wa_t, wb_t, b2d, rb):
    n, d = h.shape
    grid = (n // rb,)

    def body(h_ref, m_ref, wa_ref, wb_ref, b_ref, o_ref):
        hv = h_ref[...]
        mv = m_ref[...]
        aggr = jnp.where(mv == NEG, 0.0, mv - hv)
        acc = jnp.dot(hv, wa_ref[...], preferred_element_type=jnp.float32)
        acc = acc + jnp.dot(aggr, wb_ref[...], preferred_element_type=jnp.float32)
        o_ref[...] = jnp.maximum(acc + b_ref[...], 0.0)

    return pl.pallas_call(
        body,
        grid=grid,
        in_specs=[
            pl.BlockSpec((rb, d), lambda i: (i, 0)),
            pl.BlockSpec((rb, d), lambda i: (i, 0)),
            pl.BlockSpec((d, d), lambda i: (0, 0)),
            pl.BlockSpec((d, d), lambda i: (0, 0)),
            pl.BlockSpec((1, d), lambda i: (0, 0)),
        ],
        out_specs=pl.BlockSpec((rb, d), lambda i: (i, 0)),
        out_shape=jax.ShapeDtypeStruct((n, d), jnp.float32),
    )(h, m_pad, wa_t, wb_t, b2d)


def _pool_head(h2, batch2, w1_t, b1_2d, w2_t, b2_2d, rb):
    n, d = h2.shape
    nb = n // rb
    dh = w1_t.shape[1]
    nc = w2_t.shape[1]

    def body(h_ref, b_ref, w1_ref, b1_ref, w2_ref, b2_ref, o_ref, acc_ref):
        pid = pl.program_id(0)

        @pl.when(pid == 0)
        def _():
            acc_ref[...] = jnp.full((GRAPHS, d), NEG, jnp.float32)

        bv = b_ref[...]
        hv = h_ref[...]
        for g in range(GRAPHS):
            sel = jnp.where(bv == g, hv, NEG)
            red = jnp.max(sel, axis=0, keepdims=True)
            acc_ref[pl.ds(g, 1), :] = jnp.maximum(acc_ref[pl.ds(g, 1), :], red)

        @pl.when(pid == nb - 1)
        def _():
            pooled = acc_ref[...]
            pooled = jnp.where(pooled == NEG, 0.0, pooled)
            z1 = jnp.dot(pooled, w1_ref[...], preferred_element_type=jnp.float32)
            z1 = jnp.maximum(z1 + b1_ref[...], 0.0)
            z = jnp.dot(z1, w2_ref[...], preferred_element_type=jnp.float32)
            z = z + b2_ref[...]
            zm = jnp.max(z, axis=1, keepdims=True)
            lse = jnp.log(jnp.sum(jnp.exp(z - zm), axis=1, keepdims=True)) + zm
            o_ref[...] = z - lse

    return pl.pallas_call(
        body,
        grid=(nb,),
        in_specs=[
            pl.BlockSpec((rb, d), lambda i: (i, 0)),
            pl.BlockSpec((rb, 1), lambda i: (i, 0)),
            pl.BlockSpec((d, dh), lambda i: (0, 0)),
            pl.BlockSpec((1, dh), lambda i: (0, 0)),
            pl.BlockSpec((dh, nc), lambda i: (0, 0)),
            pl.BlockSpec((1, nc), lambda i: (0, 0)),
        ],
        out_specs=pl.BlockSpec((GRAPHS, nc), lambda i: (0, 0)),
        out_shape=jax.ShapeDtypeStruct((GRAPHS, nc), jnp.float32),
        scratch_shapes=[pltpu.VMEM((GRAPHS, d), jnp.float32)],
    )(h2, batch2, w1_t, b1_2d, w2_t, b2_2d)


def kernel(x, edge_index, batch, W1, b1, W2, b2, Wf1, bf1, Wf2, bf2):
    n, d = x.shape
    e = edge_index.shape[1]
    assert n <= IMASK + 1
    rows_w = (-(-n // NW) + 7) // 8 * 8
    n_pad = rows_w * NW
    rb = 2000

    src, dst = edge_index[0], edge_index[1]
    s2 = jnp.concatenate([src, dst])
    d2 = jnp.concatenate([dst, src])
    packed = jnp.bitwise_or(s2, d2 << SHIFT).astype(jnp.int32)
    ep = -(-2 * e // CHUNK) * CHUNK
    if ep > 2 * e:
        sentinel = jnp.full((ep - 2 * e,), IMASK << SHIFT, jnp.int32)
        packed = jnp.concatenate([packed, sentinel])

    w1a_t = W1[:, :d].T
    w1b_t = W1[:, d:].T
    w2a_t = W2[:, :d].T
    w2b_t = W2[:, d:].T

    binned, counts = _bin_edges(packed, rows_w)
    m1 = _seg_max(binned, counts, x, n_pad, rows_w, ep)
    h1 = _layer(x, m1, w1a_t, w1b_t, b1.reshape(1, d), rb)
    m2 = _seg_max(binned, counts, h1, n_pad, rows_w, ep)
    h2 = _layer(h1, m2, w2a_t, w2b_t, b2.reshape(1, d), rb)

    batch2 = batch.reshape(n, 1)
    return _pool_head(
        h2, batch2, Wf1.T, bf1.reshape(1, -1), Wf2.T, bf2.reshape(1, -1), rb
    )


# speedup vs baseline: 6.4315x; 1.2234x over previous
import jax
import jax.numpy as jnp
from jax import lax
from jax.experimental import pallas as pl
from jax.experimental.pallas import tpu as pltpu
from jax.experimental.pallas import tpu_sc as plsc

NC = 2
NS = 16
NW = NC * NS
L = 16
SHIFT = 14
IMASK = (1 << SHIFT) - 1
KB = 128
CHUNK = 2048
NEG = float("-inf")
GRAPHS = 8

_SC_PARAMS = pltpu.CompilerParams(needs_layout_passes=False)


def _popcount(m):
    return plsc.all_reduce_population_count(m)[0]


def _bin_edges(packed, rows_w):
    ep = packed.shape[0]
    n_chunks = ep // CHUNK
    cap = ep
    mesh = plsc.VectorSubcoreMesh(
        core_axis_name="c", subcore_axis_name="s", num_cores=NC, num_subcores=NS
    )

    def body(packed_hbm, binned_hbm, counts_hbm, inbuf, pbuf, cntb, sem_in):
        w = lax.axis_index("s") * NC + lax.axis_index("c")
        base = w * rows_w
        rbase = w * cap

        pltpu.async_copy(
            packed_hbm.at[pl.ds(0, CHUNK)], inbuf.at[pl.ds(0, CHUNK)], sem_in
        )

        def chunk_body(c, carry):
            p, nf = carry
            b = (c % 2) * CHUNK
            pltpu.make_async_copy(
                packed_hbm.at[pl.ds(0, CHUNK)], inbuf.at[pl.ds(b, CHUNK)], sem_in
            ).wait()

            @pl.when(c + 1 < n_chunks)
            def _():
                b2 = ((c + 1) % 2) * CHUNK
                pltpu.async_copy(
                    packed_hbm.at[pl.ds((c + 1) * CHUNK, CHUNK)],
                    inbuf.at[pl.ds(b2, CHUNK)],
                    sem_in,
                )

            rw = jnp.uint32(rows_w)

            def scan_body(i, carry):
                p, nf = carry
                v0 = inbuf[pl.ds(b + i * 2 * L, L)]
                v1 = inbuf[pl.ds(b + i * 2 * L + L, L)]
                m0 = ((v0 >> SHIFT) - base).astype(jnp.uint32) < rw
                m1 = ((v1 >> SHIFT) - base).astype(jnp.uint32) < rw
                c0 = _popcount(m0)
                c1 = _popcount(m1)
                plsc.store_compressed(pbuf.at[pl.ds(p, L)], v0, mask=m0)
                p1 = p + c0
                plsc.store_compressed(pbuf.at[pl.ds(p1, L)], v1, mask=m1)
                p = p1 + c1
                full = p >= KB

                @pl.when(full)
                def _():
                    pltpu.sync_copy(
                        pbuf.at[pl.ds(0, KB)],
                        binned_hbm.at[pl.ds(rbase + nf * KB, KB)],
                    )
                    pbuf[pl.ds(0, L)] = pbuf[pl.ds(KB, L)]
                    pbuf[pl.ds(L, L)] = pbuf[pl.ds(KB + L, L)]

                p = jnp.where(full, p - KB, p)
                nf = jnp.where(full, nf + 1, nf)
                return p, nf

            return lax.fori_loop(0, CHUNK // (2 * L), scan_body, (p, nf))

        p, nf = lax.fori_loop(0, n_chunks, chunk_body, (jnp.int32(0), jnp.int32(0)))

        pltpu.sync_copy(
            pbuf.at[pl.ds(0, KB)],
            binned_hbm.at[pl.ds(rbase + nf * KB, KB)],
        )
        count = nf * KB + p
        cntb[pl.ds(0, L)] = jnp.full((L,), 0, jnp.int32) + count
        pltpu.sync_copy(cntb.at[pl.ds(0, L)], counts_hbm.at[pl.ds(w * 128, L)])

    return pl.kernel(
        body,
        out_type=(
            jax.ShapeDtypeStruct((NW * cap,), jnp.int32),
            jax.ShapeDtypeStruct((NW * 128,), jnp.int32),
        ),
        mesh=mesh,
        compiler_params=_SC_PARAMS,
        scratch_types=[
            pltpu.VMEM((2 * CHUNK,), jnp.int32),
            pltpu.VMEM((KB + 2 * L,), jnp.int32),
            pltpu.VMEM((L,), jnp.int32),
            pltpu.SemaphoreType.DMA,
        ],
    )(packed)


def _seg_max(binned, counts, h, n_pad, rows_w, cap):
    n, d = h.shape
    nr = d // L
    mesh = plsc.VectorSubcoreMesh(
        core_axis_name="c", subcore_axis_name="s", num_cores=NC, num_subcores=NS
    )

    ns = 4

    def body(binned_hbm, counts_hbm, h_hbm, m_hbm, pbuf, idxb, cntb, rows, acc,
             sem_in, sg0, sg1, sg2, sg3):
        w = lax.axis_index("s") * NC + lax.axis_index("c")
        base = w * rows_w
        rbase = w * cap
        neg = jnp.full((L,), NEG, jnp.float32)
        sems = (sg0, sg1, sg2, sg3)

        pltpu.sync_copy(counts_hbm.at[pl.ds(w * 128, L)], cntb)
        count = cntb[pl.ds(0, L)][0]
        n_blocks = (count + KB - 1) >> 7

        def init_body(i, carry):
            for r in range(nr):
                acc[i, pl.ds(r * L, L)] = neg
            return carry

        lax.fori_loop(0, rows_w, init_body, 0)

        def prep(g, s):
            off = s * KB
            pltpu.async_copy(
                binned_hbm.at[pl.ds(rbase + g * KB, KB)],
                pbuf.at[pl.ds(off, KB)],
                sem_in,
            )
            pltpu.make_async_copy(
                binned_hbm.at[pl.ds(rbase, KB)], pbuf.at[pl.ds(off, KB)], sem_in
            ).wait()
            for j in range(KB // L):
                v = pbuf[pl.ds(off + j * L, L)]
                idxb[pl.ds(off + j * L, L)] = jnp.minimum(v & IMASK, n - 1)
            pltpu.async_copy(
                h_hbm.at[idxb.at[pl.ds(off, KB)]],
                rows.at[pl.ds(off, KB)],
                sems[s],
            )

        def wait_gather(s):
            pltpu.make_async_copy(
                h_hbm.at[idxb.at[pl.ds(s * KB, KB)]],
                rows.at[pl.ds(s * KB, KB)],
                sems[s],
            ).wait()

        def fold_full(s):
            off = s * KB

            def fold_grp(jb, carry):
                vv = pbuf[pl.ds(off + jb * L, L)]
                dlv = (vv >> SHIFT) - base
                for je in range(L):
                    dl = dlv[je]
                    e = off + jb * L + je
                    for r in range(nr):
                        sl = pl.ds(r * L, L)
                        acc[dl, sl] = jnp.maximum(acc[dl, sl], rows[e, sl])
                return carry

            lax.fori_loop(0, KB // L, fold_grp, 0)

        def fold_partial(g, s):
            nvalid = count - g * KB

            def fold_e(e, carry):
                v = pbuf[pl.ds(s * KB + e, L)][0]
                dl = (v >> SHIFT) - base
                for r in range(nr):
                    sl = pl.ds(r * L, L)
                    acc[dl, sl] = jnp.maximum(
                        acc[dl, sl], rows[s * KB + e, sl]
                    )
                return carry

            lax.fori_loop(0, nvalid, fold_e, 0)

        for k in range(ns):
            @pl.when(k < n_blocks)
            def _(k=k):
                prep(jnp.int32(k), k)

        def block_body(g, carry):
            slot = g % ns
            for s in range(ns):
                @pl.when(slot == s)
                def _(s=s):
                    wait_gather(s)

                    @pl.when(g < n_blocks - 1)
                    def _():
                        fold_full(s)

                    @pl.when(g == n_blocks - 1)
                    def _():
                        fold_partial(g, s)

                    @pl.when(g + ns < n_blocks)
                    def _():
                        prep(g + ns, s)

            return carry

        lax.fori_loop(0, n_blocks, block_body, 0)

        pltpu.sync_copy(acc, m_hbm.at[pl.ds(base, rows_w)])

    return pl.kernel(
        body,
        out_type=jax.ShapeDtypeStruct((n_pad, d), jnp.float32),
        mesh=mesh,
        compiler_params=_SC_PARAMS,
        scratch_types=[
            pltpu.VMEM((ns * KB + L,), jnp.int32),
            pltpu.VMEM((ns * KB,), jnp.int32),
            pltpu.VMEM((L,), jnp.int32),
            pltpu.VMEM((ns * KB, d), jnp.float32),
            pltpu.VMEM((rows_w, d), jnp.float32),
        ] + [pltpu.SemaphoreType.DMA] * (1 + ns),
    )(binned, counts, h)


def _layer(h, m_pad, wa_t, wb_t, b2d, rb):
    n, d = h.shape
    grid = (n // rb,)

    def body(h_ref, m_ref, wa_ref, wb_ref, b_ref, o_ref):
        hv = h_ref[...]
        mv = m_ref[...]
        aggr = jnp.where(mv == NEG, 0.0, mv - hv)
        acc = jnp.dot(hv, wa_ref[...], preferred_element_type=jnp.float32)
        acc = acc + jnp.dot(aggr, wb_ref[...], preferred_element_type=jnp.float32)
        o_ref[...] = jnp.maximum(acc + b_ref[...], 0.0)

    return pl.pallas_call(
        body,
        grid=grid,
        in_specs=[
            pl.BlockSpec((rb, d), lambda i: (i, 0)),
            pl.BlockSpec((rb, d), lambda i: (i, 0)),
            pl.BlockSpec((d, d), lambda i: (0, 0)),
            pl.BlockSpec((d, d), lambda i: (0, 0)),
            pl.BlockSpec((1, d), lambda i: (0, 0)),
        ],
        out_specs=pl.BlockSpec((rb, d), lambda i: (i, 0)),
        out_shape=jax.ShapeDtypeStruct((n, d), jnp.float32),
    )(h, m_pad, wa_t, wb_t, b2d)


def _pool_head(h2, batch2, w1_t, b1_2d, w2_t, b2_2d, rb):
    n, d = h2.shape
    nb = n // rb
    dh = w1_t.shape[1]
    nc = w2_t.shape[1]

    def body(h_ref, b_ref, w1_ref, b1_ref, w2_ref, b2_ref, o_ref, acc_ref):
        pid = pl.program_id(0)

        @pl.when(pid == 0)
        def _():
            acc_ref[...] = jnp.full((GRAPHS, d), NEG, jnp.float32)

        bv = b_ref[...]
        hv = h_ref[...]
        for g in range(GRAPHS):
            sel = jnp.where(bv == g, hv, NEG)
            red = jnp.max(sel, axis=0, keepdims=True)
            acc_ref[pl.ds(g, 1), :] = jnp.maximum(acc_ref[pl.ds(g, 1), :], red)

        @pl.when(pid == nb - 1)
        def _():
            pooled = acc_ref[...]
            pooled = jnp.where(pooled == NEG, 0.0, pooled)
            z1 = jnp.dot(pooled, w1_ref[...], preferred_element_type=jnp.float32)
            z1 = jnp.maximum(z1 + b1_ref[...], 0.0)
            z = jnp.dot(z1, w2_ref[...], preferred_element_type=jnp.float32)
            z = z + b2_ref[...]
            zm = jnp.max(z, axis=1, keepdims=True)
            lse = jnp.log(jnp.sum(jnp.exp(z - zm), axis=1, keepdims=True)) + zm
            o_ref[...] = z - lse

    return pl.pallas_call(
        body,
        grid=(nb,),
        in_specs=[
            pl.BlockSpec((rb, d), lambda i: (i, 0)),
            pl.BlockSpec((rb, 1), lambda i: (i, 0)),
            pl.BlockSpec((d, dh), lambda i: (0, 0)),
            pl.BlockSpec((1, dh), lambda i: (0, 0)),
            pl.BlockSpec((dh, nc), lambda i: (0, 0)),
            pl.BlockSpec((1, nc), lambda i: (0, 0)),
        ],
        out_specs=pl.BlockSpec((GRAPHS, nc), lambda i: (0, 0)),
        out_shape=jax.ShapeDtypeStruct((GRAPHS, nc), jnp.float32),
        scratch_shapes=[pltpu.VMEM((GRAPHS, d), jnp.float32)],
    )(h2, batch2, w1_t, b1_2d, w2_t, b2_2d)


def kernel(x, edge_index, batch, W1, b1, W2, b2, Wf1, bf1, Wf2, bf2):
    n, d = x.shape
    e = edge_index.shape[1]
    assert n <= IMASK + 1
    rows_w = (-(-n // NW) + 7) // 8 * 8
    n_pad = rows_w * NW
    rb = 2000

    src, dst = edge_index[0], edge_index[1]
    s2 = jnp.concatenate([src, dst])
    d2 = jnp.concatenate([dst, src])
    packed = jnp.bitwise_or(s2, d2 << SHIFT).astype(jnp.int32)
    ep = -(-2 * e // CHUNK) * CHUNK
    if ep > 2 * e:
        sentinel = jnp.full((ep - 2 * e,), IMASK << SHIFT, jnp.int32)
        packed = jnp.concatenate([packed, sentinel])

    w1a_t = W1[:, :d].T
    w1b_t = W1[:, d:].T
    w2a_t = W2[:, :d].T
    w2b_t = W2[:, d:].T

    binned, counts = _bin_edges(packed, rows_w)
    m1 = _seg_max(binned, counts, x, n_pad, rows_w, ep)
    h1 = _layer(x, m1, w1a_t, w1b_t, b1.reshape(1, d), rb)
    m2 = _seg_max(binned, counts, h1, n_pad, rows_w, ep)
    h2 = _layer(h1, m2, w2a_t, w2b_t, b2.reshape(1, d), rb)

    batch2 = batch.reshape(n, 1)
    return _pool_head(
        h2, batch2, Wf1.T, bf1.reshape(1, -1), Wf2.T, bf2.reshape(1, -1), rb
    )


# speedup vs baseline: 6.6454x; 1.0333x over previous
import jax
import jax.numpy as jnp
from jax import lax
from jax.experimental import pallas as pl
from jax.experimental.pallas import tpu as pltpu
from jax.experimental.pallas import tpu_sc as plsc

NC = 2
NS = 16
NW = NC * NS
L = 16
SHIFT = 14
IMASK = (1 << SHIFT) - 1
KB = 128
CHUNK = 2048
NEG = float("-inf")
GRAPHS = 8

_SC_PARAMS = pltpu.CompilerParams(needs_layout_passes=False)


def _popcount(m):
    return plsc.all_reduce_population_count(m)[0]


def _bin_edges(packed, rows_w):
    ep = packed.shape[0]
    n_chunks = ep // CHUNK
    cap = ep
    mesh = plsc.VectorSubcoreMesh(
        core_axis_name="c", subcore_axis_name="s", num_cores=NC, num_subcores=NS
    )

    def body(packed_hbm, binned_hbm, counts_hbm, inbuf, pbuf, cntb, sem_in):
        w = lax.axis_index("s") * NC + lax.axis_index("c")
        base = w * rows_w
        rbase = w * cap

        pltpu.async_copy(
            packed_hbm.at[pl.ds(0, CHUNK)], inbuf.at[pl.ds(0, CHUNK)], sem_in
        )

        def chunk_body(c, carry):
            p, nf = carry
            b = (c % 2) * CHUNK
            pltpu.make_async_copy(
                packed_hbm.at[pl.ds(0, CHUNK)], inbuf.at[pl.ds(b, CHUNK)], sem_in
            ).wait()

            @pl.when(c + 1 < n_chunks)
            def _():
                b2 = ((c + 1) % 2) * CHUNK
                pltpu.async_copy(
                    packed_hbm.at[pl.ds((c + 1) * CHUNK, CHUNK)],
                    inbuf.at[pl.ds(b2, CHUNK)],
                    sem_in,
                )

            rw = jnp.uint32(rows_w)

            def scan_body(i, carry):
                p, nf = carry
                v0 = inbuf[pl.ds(b + i * 2 * L, L)]
                v1 = inbuf[pl.ds(b + i * 2 * L + L, L)]
                m0 = ((v0 >> SHIFT) - base).astype(jnp.uint32) < rw
                m1 = ((v1 >> SHIFT) - base).astype(jnp.uint32) < rw
                c0 = _popcount(m0)
                c1 = _popcount(m1)
                plsc.store_compressed(pbuf.at[pl.ds(p, L)], v0, mask=m0)
                p1 = p + c0
                plsc.store_compressed(pbuf.at[pl.ds(p1, L)], v1, mask=m1)
                p = p1 + c1
                full = p >= KB

                @pl.when(full)
                def _():
                    pltpu.sync_copy(
                        pbuf.at[pl.ds(0, KB)],
                        binned_hbm.at[pl.ds(rbase + nf * KB, KB)],
                    )
                    pbuf[pl.ds(0, L)] = pbuf[pl.ds(KB, L)]
                    pbuf[pl.ds(L, L)] = pbuf[pl.ds(KB + L, L)]

                p = jnp.where(full, p - KB, p)
                nf = jnp.where(full, nf + 1, nf)
                return p, nf

            return lax.fori_loop(0, CHUNK // (2 * L), scan_body, (p, nf))

        p, nf = lax.fori_loop(0, n_chunks, chunk_body, (jnp.int32(0), jnp.int32(0)))

        pltpu.sync_copy(
            pbuf.at[pl.ds(0, KB)],
            binned_hbm.at[pl.ds(rbase + nf * KB, KB)],
        )
        count = nf * KB + p
        cntb[pl.ds(0, L)] = jnp.full((L,), 0, jnp.int32) + count
        pltpu.sync_copy(cntb.at[pl.ds(0, L)], counts_hbm.at[pl.ds(w * 128, L)])

    return pl.kernel(
        body,
        out_type=(
            jax.ShapeDtypeStruct((NW * cap,), jnp.int32),
            jax.ShapeDtypeStruct((NW * 128,), jnp.int32),
        ),
        mesh=mesh,
        compiler_params=_SC_PARAMS,
        scratch_types=[
            pltpu.VMEM((2 * CHUNK,), jnp.int32),
            pltpu.VMEM((KB + 2 * L,), jnp.int32),
            pltpu.VMEM((L,), jnp.int32),
            pltpu.SemaphoreType.DMA,
        ],
    )(packed)


def _bin_seg_max(packed, h, n_pad, rows_w):
    ep = packed.shape[0]
    n_chunks = ep // CHUNK
    cap = ep
    n, d = h.shape
    nr = d // L
    nrg = 4
    mesh = plsc.VectorSubcoreMesh(
        core_axis_name="c", subcore_axis_name="s", num_cores=NC, num_subcores=NS
    )

    def body(packed_hbm, h_hbm, binned_hbm, counts_hbm, m_hbm,
             inbuf, pbuf, pkeep, idxb, cntb, rows, acc,
             sem_in, sp0, sp1, sp2, sp3, sg0, sg1, sg2, sg3):
        w = lax.axis_index("s") * NC + lax.axis_index("c")
        base = w * rows_w
        rbase = w * cap
        neg = jnp.full((L,), NEG, jnp.float32)
        spills = (sp0, sp1, sp2, sp3)
        gathers = (sg0, sg1, sg2, sg3)

        def init_body(i, carry):
            for r in range(nr):
                acc[i, pl.ds(r * L, L)] = neg
            return carry

        lax.fori_loop(0, rows_w, init_body, 0)

        def wait_gather(s):
            pltpu.make_async_copy(
                h_hbm.at[idxb.at[pl.ds(0, KB)]],
                rows.at[pl.ds(0, KB)],
                gathers[s],
            ).wait()

        def wait_spill(s):
            pltpu.make_async_copy(
                pkeep.at[pl.ds(0, KB)],
                binned_hbm.at[pl.ds(rbase, KB)],
                spills[s],
            ).wait()

        def fold_full(off):
            def fold_grp(jb, carry):
                vv = pkeep[pl.ds(off + jb * L, L)]
                dlv = (vv >> SHIFT) - base
                for je in range(L):
                    dl = dlv[je]
                    e = off + jb * L + je
                    for r in range(nr):
                        sl = pl.ds(r * L, L)
                        acc[dl, sl] = jnp.maximum(acc[dl, sl], rows[e, sl])
                return carry

            lax.fori_loop(0, KB // L, fold_grp, 0)

        def stash(off):
            for j in range(KB // L):
                vv = pbuf[pl.ds(j * L, L)]
                pkeep[pl.ds(off + j * L, L)] = vv
                idxb[pl.ds(off + j * L, L)] = jnp.minimum(vv & IMASK, n - 1)

        def fire(s, off, nf):
            pltpu.async_copy(
                pkeep.at[pl.ds(off, KB)],
                binned_hbm.at[pl.ds(rbase + nf * KB, KB)],
                spills[s],
            )
            pltpu.async_copy(
                h_hbm.at[idxb.at[pl.ds(off, KB)]],
                rows.at[pl.ds(off, KB)],
                gathers[s],
            )

        pltpu.async_copy(
            packed_hbm.at[pl.ds(0, CHUNK)], inbuf.at[pl.ds(0, CHUNK)], sem_in
        )

        def chunk_body(c, carry):
            p, nf = carry
            b = (c % 2) * CHUNK
            pltpu.make_async_copy(
                packed_hbm.at[pl.ds(0, CHUNK)], inbuf.at[pl.ds(b, CHUNK)], sem_in
            ).wait()

            @pl.when(c + 1 < n_chunks)
            def _():
                b2 = ((c + 1) % 2) * CHUNK
                pltpu.async_copy(
                    packed_hbm.at[pl.ds((c + 1) * CHUNK, CHUNK)],
                    inbuf.at[pl.ds(b2, CHUNK)],
                    sem_in,
                )

            rw = jnp.uint32(rows_w)

            def scan_body(i, carry):
                p, nf = carry
                v0 = inbuf[pl.ds(b + i * 2 * L, L)]
                v1 = inbuf[pl.ds(b + i * 2 * L + L, L)]
                m0 = ((v0 >> SHIFT) - base).astype(jnp.uint32) < rw
                m1 = ((v1 >> SHIFT) - base).astype(jnp.uint32) < rw
                c0 = _popcount(m0)
                c1 = _popcount(m1)
                plsc.store_compressed(pbuf.at[pl.ds(p, L)], v0, mask=m0)
                p1 = p + c0
                plsc.store_compressed(pbuf.at[pl.ds(p1, L)], v1, mask=m1)
                p = p1 + c1
                full = p >= KB

                @pl.when(full)
                def _():
                    slot = nf % nrg
                    off = slot * KB

                    @pl.when(nf >= nrg)
                    def _():
                        for s in range(nrg):
                            @pl.when(slot == s)
                            def _(s=s):
                                wait_gather(s)

                        fold_full(off)
                        for s in range(nrg):
                            @pl.when(slot == s)
                            def _(s=s):
                                wait_spill(s)

                    stash(off)
                    for s in range(nrg):
                        @pl.when(slot == s)
                        def _(s=s):
                            fire(s, off, nf)

                    pbuf[pl.ds(0, L)] = pbuf[pl.ds(KB, L)]
                    pbuf[pl.ds(L, L)] = pbuf[pl.ds(KB + L, L)]

                p = jnp.where(full, p - KB, p)
                nf = jnp.where(full, nf + 1, nf)
                return p, nf

            return lax.fori_loop(0, CHUNK // (2 * L), scan_body, (p, nf))

        p, nf = lax.fori_loop(0, n_chunks, chunk_body, (jnp.int32(0), jnp.int32(0)))

        def drain_body(k, carry):
            jdx = nf - nrg + k

            @pl.when(jdx >= 0)
            def _():
                slot = jdx % nrg
                for s in range(nrg):
                    @pl.when(slot == s)
                    def _(s=s):
                        wait_gather(s)

                fold_full(slot * KB)
                for s in range(nrg):
                    @pl.when(slot == s)
                    def _(s=s):
                        wait_spill(s)

            return carry

        lax.fori_loop(0, nrg, drain_body, 0)

        pltpu.sync_copy(
            pbuf.at[pl.ds(0, KB)],
            binned_hbm.at[pl.ds(rbase + nf * KB, KB)],
        )
        lanes = lax.iota(jnp.int32, L)
        for j in range(KB // L):
            gid = lanes + j * L
            vv = pbuf[pl.ds(j * L, L)]
            idxb[pl.ds(j * L, L)] = jnp.where(
                gid < p, jnp.minimum(vv & IMASK, n - 1), 0
            )
            pkeep[pl.ds(j * L, L)] = vv
        pltpu.async_copy(
            h_hbm.at[idxb.at[pl.ds(0, KB)]], rows.at[pl.ds(0, KB)], sg0
        )
        pltpu.make_async_copy(
            h_hbm.at[idxb.at[pl.ds(0, KB)]], rows.at[pl.ds(0, KB)], sg0
        ).wait()

        def fold_e(e, carry):
            v = pkeep[pl.ds(e, L)][0]
            dl = (v >> SHIFT) - base
            for r in range(nr):
                sl = pl.ds(r * L, L)
                acc[dl, sl] = jnp.maximum(acc[dl, sl], rows[e, sl])
            return carry

        lax.fori_loop(0, p, fold_e, 0)

        count = nf * KB + p
        cntb[pl.ds(0, L)] = jnp.full((L,), 0, jnp.int32) + count
        pltpu.sync_copy(cntb.at[pl.ds(0, L)], counts_hbm.at[pl.ds(w * 128, L)])
        pltpu.sync_copy(acc, m_hbm.at[pl.ds(base, rows_w)])

    return pl.kernel(
        body,
        out_type=(
            jax.ShapeDtypeStruct((NW * cap,), jnp.int32),
            jax.ShapeDtypeStruct((NW * 128,), jnp.int32),
            jax.ShapeDtypeStruct((n_pad, d), jnp.float32),
        ),
        mesh=mesh,
        compiler_params=_SC_PARAMS,
        scratch_types=[
            pltpu.VMEM((2 * CHUNK,), jnp.int32),
            pltpu.VMEM((KB + 2 * L,), jnp.int32),
            pltpu.VMEM((nrg * KB + L,), jnp.int32),
            pltpu.VMEM((nrg * KB,), jnp.int32),
            pltpu.VMEM((L,), jnp.int32),
            pltpu.VMEM((nrg * KB, d), jnp.float32),
            pltpu.VMEM((rows_w, d), jnp.float32),
        ] + [pltpu.SemaphoreType.DMA] * 9,
    )(packed, h)


def _seg_max(binned, counts, h, n_pad, rows_w, cap):
    n, d = h.shape
    nr = d // L
    mesh = plsc.VectorSubcoreMesh(
        core_axis_name="c", subcore_axis_name="s", num_cores=NC, num_subcores=NS
    )

    ns = 4

    def body(binned_hbm, counts_hbm, h_hbm, m_hbm, pbuf, idxb, cntb, rows, acc,
             sem_in, sg0, sg1, sg2, sg3):
        w = lax.axis_index("s") * NC + lax.axis_index("c")
        base = w * rows_w
        rbase = w * cap
        neg = jnp.full((L,), NEG, jnp.float32)
        sems = (sg0, sg1, sg2, sg3)

        pltpu.sync_copy(counts_hbm.at[pl.ds(w * 128, L)], cntb)
        count = cntb[pl.ds(0, L)][0]
        n_blocks = (count + KB - 1) >> 7

        def init_body(i, carry):
            for r in range(nr):
                acc[i, pl.ds(r * L, L)] = neg
            return carry

        lax.fori_loop(0, rows_w, init_body, 0)

        def prep(g, s):
            off = s * KB
            pltpu.async_copy(
                binned_hbm.at[pl.ds(rbase + g * KB, KB)],
                pbuf.at[pl.ds(off, KB)],
                sem_in,
            )
            pltpu.make_async_copy(
                binned_hbm.at[pl.ds(rbase, KB)], pbuf.at[pl.ds(off, KB)], sem_in
            ).wait()
            for j in range(KB // L):
                v = pbuf[pl.ds(off + j * L, L)]
                idxb[pl.ds(off + j * L, L)] = jnp.minimum(v & IMASK, n - 1)
            pltpu.async_copy(
                h_hbm.at[idxb.at[pl.ds(off, KB)]],
                rows.at[pl.ds(off, KB)],
                sems[s],
            )

        def wait_gather(s):
            pltpu.make_async_copy(
                h_hbm.at[idxb.at[pl.ds(s * KB, KB)]],
                rows.at[pl.ds(s * KB, KB)],
                sems[s],
            ).wait()

        def fold_full(s):
            off = s * KB

            def fold_grp(jb, carry):
                vv = pbuf[pl.ds(off + jb * L, L)]
                dlv = (vv >> SHIFT) - base
                for je in range(L):
                    dl = dlv[je]
                    e = off + jb * L + je
                    for r in range(nr):
                        sl = pl.ds(r * L, L)
                        acc[dl, sl] = jnp.maximum(acc[dl, sl], rows[e, sl])
                return carry

            lax.fori_loop(0, KB // L, fold_grp, 0)

        def fold_partial(g, s):
            nvalid = count - g * KB

            def fold_e(e, carry):
                v = pbuf[pl.ds(s * KB + e, L)][0]
                dl = (v >> SHIFT) - base
                for r in range(nr):
                    sl = pl.ds(r * L, L)
                    acc[dl, sl] = jnp.maximum(
                        acc[dl, sl], rows[s * KB + e, sl]
                    )
                return carry

            lax.fori_loop(0, nvalid, fold_e, 0)

        for k in range(ns):
            @pl.when(k < n_blocks)
            def _(k=k):
                prep(jnp.int32(k), k)

        def block_body(g, carry):
            slot = g % ns
            for s in range(ns):
                @pl.when(slot == s)
                def _(s=s):
                    wait_gather(s)

                    @pl.when(g < n_blocks - 1)
                    def _():
                        fold_full(s)

                    @pl.when(g == n_blocks - 1)
                    def _():
                        fold_partial(g, s)

                    @pl.when(g + ns < n_blocks)
                    def _():
                        prep(g + ns, s)

            return carry

        lax.fori_loop(0, n_blocks, block_body, 0)

        pltpu.sync_copy(acc, m_hbm.at[pl.ds(base, rows_w)])

    return pl.kernel(
        body,
        out_type=jax.ShapeDtypeStruct((n_pad, d), jnp.float32),
        mesh=mesh,
        compiler_params=_SC_PARAMS,
        scratch_types=[
            pltpu.VMEM((ns * KB + L,), jnp.int32),
            pltpu.VMEM((ns * KB,), jnp.int32),
            pltpu.VMEM((L,), jnp.int32),
            pltpu.VMEM((ns * KB, d), jnp.float32),
            pltpu.VMEM((rows_w, d), jnp.float32),
        ] + [pltpu.SemaphoreType.DMA] * (1 + ns),
    )(binned, counts, h)


def _layer(h, m_pad, wa_t, wb_t, b2d, rb):
    n, d = h.shape
    grid = (n // rb,)

    def body(h_ref, m_ref, wa_ref, wb_ref, b_ref, o_ref):
        hv = h_ref[...]
        mv = m_ref[...]
        aggr = jnp.where(mv == NEG, 0.0, mv - hv)
        acc = jnp.dot(hv, wa_ref[...], preferred_element_type=jnp.float32)
        acc = acc + jnp.dot(aggr, wb_ref[...], preferred_element_type=jnp.float32)
        o_ref[...] = jnp.maximum(acc + b_ref[...], 0.0)

    return pl.pallas_call(
        body,
        grid=grid,
        in_specs=[
            pl.BlockSpec((rb, d), lambda i: (i, 0)),
            pl.BlockSpec((rb, d), lambda i: (i, 0)),
            pl.BlockSpec((d, d), lambda i: (0, 0)),
            pl.BlockSpec((d, d), lambda i: (0, 0)),
            pl.BlockSpec((1, d), lambda i: (0, 0)),
        ],
        out_specs=pl.BlockSpec((rb, d), lambda i: (i, 0)),
        out_shape=jax.ShapeDtypeStruct((n, d), jnp.float32),
    )(h, m_pad, wa_t, wb_t, b2d)


def _pool_head(h2, batch2, w1_t, b1_2d, w2_t, b2_2d, rb):
    n, d = h2.shape
    nb = n // rb
    dh = w1_t.shape[1]
    nc = w2_t.shape[1]

    def body(h_ref, b_ref, w1_ref, b1_ref, w2_ref, b2_ref, o_ref, acc_ref):
        pid = pl.program_id(0)

        @pl.when(pid == 0)
        def _():
            acc_ref[...] = jnp.full((GRAPHS, d), NEG, jnp.float32)

        bv = b_ref[...]
        hv = h_ref[...]
        for g in range(GRAPHS):
            sel = jnp.where(bv == g, hv, NEG)
            red = jnp.max(sel, axis=0, keepdims=True)
            acc_ref[pl.ds(g, 1), :] = jnp.maximum(acc_ref[pl.ds(g, 1), :], red)

        @pl.when(pid == nb - 1)
        def _():
            pooled = acc_ref[...]
            pooled = jnp.where(pooled == NEG, 0.0, pooled)
            z1 = jnp.dot(pooled, w1_ref[...], preferred_element_type=jnp.float32)
            z1 = jnp.maximum(z1 + b1_ref[...], 0.0)
            z = jnp.dot(z1, w2_ref[...], preferred_element_type=jnp.float32)
            z = z + b2_ref[...]
            zm = jnp.max(z, axis=1, keepdims=True)
            lse = jnp.log(jnp.sum(jnp.exp(z - zm), axis=1, keepdims=True)) + zm
            o_ref[...] = z - lse

    return pl.pallas_call(
        body,
        grid=(nb,),
        in_specs=[
            pl.BlockSpec((rb, d), lambda i: (i, 0)),
            pl.BlockSpec((rb, 1), lambda i: (i, 0)),
            pl.BlockSpec((d, dh), lambda i: (0, 0)),
            pl.BlockSpec((1, dh), lambda i: (0, 0)),
            pl.BlockSpec((dh, nc), lambda i: (0, 0)),
            pl.BlockSpec((1, nc), lambda i: (0, 0)),
        ],
        out_specs=pl.BlockSpec((GRAPHS, nc), lambda i: (0, 0)),
        out_shape=jax.ShapeDtypeStruct((GRAPHS, nc), jnp.float32),
        scratch_shapes=[pltpu.VMEM((GRAPHS, d), jnp.float32)],
    )(h2, batch2, w1_t, b1_2d, w2_t, b2_2d)


def kernel(x, edge_index, batch, W1, b1, W2, b2, Wf1, bf1, Wf2, bf2):
    n, d = x.shape
    e = edge_index.shape[1]
    assert n <= IMASK + 1
    rows_w = (-(-n // NW) + 7) // 8 * 8
    n_pad = rows_w * NW
    rb = 2000

    src, dst = edge_index[0], edge_index[1]
    s2 = jnp.concatenate([src, dst])
    d2 = jnp.concatenate([dst, src])
    packed = jnp.bitwise_or(s2, d2 << SHIFT).astype(jnp.int32)
    ep = -(-2 * e // CHUNK) * CHUNK
    if ep > 2 * e:
        sentinel = jnp.full((ep - 2 * e,), IMASK << SHIFT, jnp.int32)
        packed = jnp.concatenate([packed, sentinel])

    w1a_t = W1[:, :d].T
    w1b_t = W1[:, d:].T
    w2a_t = W2[:, :d].T
    w2b_t = W2[:, d:].T

    binned, counts, m1 = _bin_seg_max(packed, x, n_pad, rows_w)
    h1 = _layer(x, m1, w1a_t, w1b_t, b1.reshape(1, d), rb)
    m2 = _seg_max(binned, counts, h1, n_pad, rows_w, ep)
    h2 = _layer(h1, m2, w2a_t, w2b_t, b2.reshape(1, d), rb)

    batch2 = batch.reshape(n, 1)
    return _pool_head(
        h2, batch2, Wf1.T, bf1.reshape(1, -1), Wf2.T, bf2.reshape(1, -1), rb
    )
